# Initial kernel scaffold; baseline (speedup 1.0000x reference)
import jax
import jax.numpy as jnp
from jax.experimental import pallas as pl


def kernel(x, pos_edge_index, neg_edge_index, W_sgc, b_sgc, W1, b1, W2, b2, W3, b3):
    raise NotImplementedError("write your pallas kernel here")


# speedup vs baseline: 2.2441x; 2.2441x over previous
import jax
import jax.numpy as jnp
from jax import lax
from jax.experimental import pallas as pl
from jax.experimental.pallas import tpu as pltpu
from jax.experimental.pallas import tpu_sc as plsc

N = 10000
NP = 10240
D = 128
E = 320000
EP = 327680
EB = EP // 128
NW = 32
NBW = EB // NW
RS = NP // 16

_mesh = plsc.VectorSubcoreMesh(core_axis_name="c", subcore_axis_name="s")
_f32 = jnp.float32


def _sc_degree(dst2d):
    @pl.kernel(
        out_type=jax.ShapeDtypeStruct((2, NP), _f32),
        mesh=_mesh,
        scratch_types=[
            pltpu.VMEM((RS,), _f32),
            pltpu.VMEM((NBW, 128), jnp.int32),
            pltpu.VMEM_SHARED((NP,), _f32),
        ],
    )
    def k(dst_hbm, out_hbm, ones_v, idx_v, acc_sh):
        c = lax.axis_index("c")
        s = lax.axis_index("s")
        w = c * 16 + s

        @pl.loop(0, RS // 16)
        def _(i):
            ones_v[pl.ds(i * 16, 16)] = jnp.ones((16,), _f32)

        pltpu.sync_copy(ones_v, acc_sh.at[pl.ds(s * RS, RS)])
        pltpu.sync_copy(dst_hbm.at[pl.ds(w * NBW, NBW)], idx_v)
        plsc.subcore_barrier()

        @pl.loop(0, NBW)
        def _(j):
            pltpu.sync_copy(ones_v.at[pl.ds(0, 128)], acc_sh.at[idx_v.at[j]],
                            add=True)

        plsc.subcore_barrier()
        pltpu.sync_copy(acc_sh.at[pl.ds(s * RS, RS)],
                        out_hbm.at[c].at[pl.ds(s * RS, RS)])

    return k(dst2d)


def _sc_hop(g, src2d, dst2d):
    @pl.kernel(
        out_type=jax.ShapeDtypeStruct((2, NP, D), _f32),
        mesh=_mesh,
        scratch_types=[
            pltpu.VMEM((NBW, 128), jnp.int32),
            pltpu.VMEM((NBW, 128), jnp.int32),
            pltpu.VMEM((128, D), _f32),
            pltpu.VMEM_SHARED((NP, D), _f32),
        ],
    )
    def k(g_hbm, src_hbm, dst_hbm, out_hbm, isrc_v, idst_v, buf, acc_sh):
        c = lax.axis_index("c")
        s = lax.axis_index("s")
        w = c * 16 + s

        pltpu.sync_copy(g_hbm.at[pl.ds(s * RS, RS)],
                        acc_sh.at[pl.ds(s * RS, RS)])
        pltpu.sync_copy(src_hbm.at[pl.ds(w * NBW, NBW)], isrc_v)
        pltpu.sync_copy(dst_hbm.at[pl.ds(w * NBW, NBW)], idst_v)
        plsc.subcore_barrier()

        @pl.loop(0, NBW)
        def _(j):
            pltpu.sync_copy(g_hbm.at[isrc_v.at[j]], buf)
            pltpu.sync_copy(buf, acc_sh.at[idst_v.at[j]], add=True)

        plsc.subcore_barrier()
        pltpu.sync_copy(acc_sh.at[pl.ds(s * RS, RS)],
                        out_hbm.at[c].at[pl.ds(s * RS, RS)])

    return k(g, src2d, dst2d)


def _sc_pair_gather(h, src2d, dst2d):
    @pl.kernel(
        out_type=(jax.ShapeDtypeStruct((EP, D), _f32),
                  jax.ShapeDtypeStruct((EP, D), _f32)),
        mesh=_mesh,
        scratch_types=[
            pltpu.VMEM((NBW, 128), jnp.int32),
            pltpu.VMEM((NBW, 128), jnp.int32),
            pltpu.VMEM((128, D), _f32),
            pltpu.VMEM((128, D), _f32),
        ],
    )
    def k(h_hbm, src_hbm, dst_hbm, hs_hbm, hd_hbm, isrc_v, idst_v, bs, bd):
        c = lax.axis_index("c")
        s = lax.axis_index("s")
        w = c * 16 + s

        pltpu.sync_copy(src_hbm.at[pl.ds(w * NBW, NBW)], isrc_v)
        pltpu.sync_copy(dst_hbm.at[pl.ds(w * NBW, NBW)], idst_v)

        @pl.loop(0, NBW)
        def _(j):
            row0 = (w * NBW + j) * 128
            pltpu.sync_copy(h_hbm.at[isrc_v.at[j]], bs)
            pltpu.sync_copy(bs, hs_hbm.at[pl.ds(row0, 128)])
            pltpu.sync_copy(h_hbm.at[idst_v.at[j]], bd)
            pltpu.sync_copy(bd, hd_hbm.at[pl.ds(row0, 128)])

    return k(h, src2d, dst2d)


_R = 2048


def _tc_scale0(xp, degp):
    def body(x_ref, d_ref, o_ref):
        deg = d_ref[0] + d_ref[1] - 1.0
        o_ref[...] = x_ref[...] * lax.rsqrt(deg)

    return pl.pallas_call(
        body,
        grid=(NP // _R,),
        in_specs=[pl.BlockSpec((_R, D), lambda i: (i, 0)),
                  pl.BlockSpec((2, _R, 1), lambda i: (0, i, 0))],
        out_specs=pl.BlockSpec((_R, D), lambda i: (i, 0)),
        out_shape=jax.ShapeDtypeStruct((NP, D), _f32),
    )(xp, degp)


def _tc_combine_mid(p, g, degp):
    def body(p_ref, g_ref, d_ref, o_ref):
        deg = d_ref[0] + d_ref[1] - 1.0
        o_ref[...] = (p_ref[0] + p_ref[1] - g_ref[...]) / deg

    return pl.pallas_call(
        body,
        grid=(NP // _R,),
        in_specs=[pl.BlockSpec((2, _R, D), lambda i: (0, i, 0)),
                  pl.BlockSpec((_R, D), lambda i: (i, 0)),
                  pl.BlockSpec((2, _R, 1), lambda i: (0, i, 0))],
        out_specs=pl.BlockSpec((_R, D), lambda i: (i, 0)),
        out_shape=jax.ShapeDtypeStruct((NP, D), _f32),
    )(p, g, degp)


def _tc_final(p, g, degp, W_sgc, b_sgc):
    def body(p_ref, g_ref, d_ref, w_ref, b_ref, o_ref):
        deg = d_ref[0] + d_ref[1] - 1.0
        hpre = (p_ref[0] + p_ref[1] - g_ref[...]) * lax.rsqrt(deg)
        o_ref[...] = jnp.dot(hpre, w_ref[...],
                             preferred_element_type=_f32) + b_ref[...]

    return pl.pallas_call(
        body,
        grid=(NP // _R,),
        in_specs=[pl.BlockSpec((2, _R, D), lambda i: (0, i, 0)),
                  pl.BlockSpec((_R, D), lambda i: (i, 0)),
                  pl.BlockSpec((2, _R, 1), lambda i: (0, i, 0)),
                  pl.BlockSpec((D, D), lambda i: (0, 0)),
                  pl.BlockSpec((1, D), lambda i: (0, 0))],
        out_specs=pl.BlockSpec((_R, D), lambda i: (i, 0)),
        out_shape=jax.ShapeDtypeStruct((NP, D), _f32),
    )(p, g, degp, W_sgc, b_sgc)


def _tc_mlp(hs, hd, W1, b1, W2, b2, W3, b3):
    def body(hs_ref, hd_ref, w1_ref, b1_ref, w2_ref, b2_ref, w3_ref, b3_ref,
             o_ref):
        z = hs_ref[...] * hd_ref[...]
        z = jnp.maximum(
            jnp.dot(z, w1_ref[...], preferred_element_type=_f32) + b1_ref[...],
            0.0)
        z = jnp.maximum(
            jnp.dot(z, w2_ref[...], preferred_element_type=_f32) + b2_ref[...],
            0.0)
        o_ref[...] = (jnp.dot(z, w3_ref[...], preferred_element_type=_f32)
                      + b3_ref[0, 0])

    return pl.pallas_call(
        body,
        grid=(EP // _R,),
        in_specs=[pl.BlockSpec((_R, D), lambda i: (i, 0)),
                  pl.BlockSpec((_R, D), lambda i: (i, 0)),
                  pl.BlockSpec((D, D), lambda i: (0, 0)),
                  pl.BlockSpec((1, D), lambda i: (0, 0)),
                  pl.BlockSpec((D, D), lambda i: (0, 0)),
                  pl.BlockSpec((1, D), lambda i: (0, 0)),
                  pl.BlockSpec((D, 1), lambda i: (0, 0)),
                  pl.BlockSpec((1, 1), lambda i: (0, 0))],
        out_specs=pl.BlockSpec((_R, 1), lambda i: (i, 0)),
        out_shape=jax.ShapeDtypeStruct((EP, 1), _f32),
    )(hs, hd, W1, b1, W2, b2, W3, b3)


def _pad_idx(row, fill):
    pad = jnp.full((EP - E,), fill, jnp.int32)
    return jnp.concatenate([row, pad]).reshape(EB, 128)


def kernel(x, pos_edge_index, neg_edge_index, W_sgc, b_sgc, W1, b1, W2, b2,
           W3, b3):
    xp = jnp.pad(x, ((0, NP - N), (0, 0)))
    psrc = _pad_idx(pos_edge_index[0], 0)
    pdst = _pad_idx(pos_edge_index[1], N)
    nsrc = _pad_idx(neg_edge_index[0], 0)
    ndst = _pad_idx(neg_edge_index[1], N)

    degp = _sc_degree(pdst).reshape(2, NP, 1)
    g = _tc_scale0(xp, degp)
    h = None
    for hop in range(3):
        p = _sc_hop(g, psrc, pdst)
        if hop < 2:
            g = _tc_combine_mid(p, g, degp)
        else:
            h = _tc_final(p, g, degp, W_sgc, b_sgc.reshape(1, D))

    hs_p, hd_p = _sc_pair_gather(h, psrc, pdst)
    hs_n, hd_n = _sc_pair_gather(h, nsrc, ndst)
    b1r, b2r, b3r = b1.reshape(1, D), b2.reshape(1, D), b3.reshape(1, 1)
    op = _tc_mlp(hs_p, hd_p, W1, b1r, W2, b2r, W3, b3r)
    on = _tc_mlp(hs_n, hd_n, W1, b1r, W2, b2r, W3, b3r)
    return (op[:E], on[:E])


# speedup vs baseline: 2.9028x; 1.2936x over previous
import jax
import jax.numpy as jnp
from jax import lax
from jax.experimental import pallas as pl
from jax.experimental.pallas import tpu as pltpu
from jax.experimental.pallas import tpu_sc as plsc

N = 10000
NP = 10240
D = 128
E = 320000
EP = 327680
EB = EP // 128
NW = 32
NBW = EB // NW
EB64 = EP // 64
NBW64 = EB64 // NW
RS = NP // 16

_mesh = plsc.VectorSubcoreMesh(core_axis_name="c", subcore_axis_name="s")
_f32 = jnp.float32


def _sc_degree(dst2d):
    @pl.kernel(
        out_type=jax.ShapeDtypeStruct((2, NP), _f32),
        mesh=_mesh,
        scratch_types=[
            pltpu.VMEM((RS,), _f32),
            pltpu.VMEM((NBW, 128), jnp.int32),
            pltpu.VMEM_SHARED((NP,), _f32),
        ],
    )
    def k(dst_hbm, out_hbm, ones_v, idx_v, acc_sh):
        c = lax.axis_index("c")
        s = lax.axis_index("s")
        w = c * 16 + s

        @pl.loop(0, RS // 16)
        def _(i):
            ones_v[pl.ds(i * 16, 16)] = jnp.ones((16,), _f32)

        pltpu.sync_copy(ones_v, acc_sh.at[pl.ds(s * RS, RS)])
        pltpu.sync_copy(dst_hbm.at[pl.ds(w * NBW, NBW)], idx_v)
        plsc.subcore_barrier()

        @pl.loop(0, NBW)
        def _(j):
            pltpu.sync_copy(ones_v.at[pl.ds(0, 128)], acc_sh.at[idx_v.at[j]],
                            add=True)

        plsc.subcore_barrier()
        pltpu.sync_copy(acc_sh.at[pl.ds(s * RS, RS)],
                        out_hbm.at[c].at[pl.ds(s * RS, RS)])

    return k(dst2d)


def _sc_hop(g, src2d, dst2d):
    @pl.kernel(
        out_type=jax.ShapeDtypeStruct((2, NP, D), _f32),
        mesh=_mesh,
        scratch_types=[
            pltpu.VMEM((NBW64 // 2, 64), jnp.int32),
            pltpu.VMEM((NBW64 // 2, 64), jnp.int32),
            pltpu.VMEM((64, D), _f32),
            pltpu.VMEM((64, D), _f32),
            pltpu.SemaphoreType.DMA,
            pltpu.SemaphoreType.DMA,
            pltpu.VMEM_SHARED((NP, D), _f32),
        ],
    )
    def k(g_hbm, src_hbm, dst_hbm, out_hbm, isrc_v, idst_v, buf0, buf1,
          sem0, sem1, acc_sh):
        c = lax.axis_index("c")
        s = lax.axis_index("s")
        w = c * 16 + s
        half = NBW64 // 2

        pltpu.sync_copy(g_hbm.at[pl.ds(s * RS, RS)],
                        acc_sh.at[pl.ds(s * RS, RS)])
        plsc.subcore_barrier()

        dummy = g_hbm.at[pl.ds(0, 64)]
        for phase in range(2):
            base = w * NBW64 + phase * half
            pltpu.sync_copy(src_hbm.at[pl.ds(base, half)], isrc_v)
            pltpu.sync_copy(dst_hbm.at[pl.ds(base, half)], idst_v)
            pltpu.async_copy(g_hbm.at[isrc_v.at[0]], buf0, sem0)

            @pl.loop(0, half, step=2)
            def _(j):
                pltpu.make_async_copy(dummy, buf0, sem0).wait()
                pltpu.async_copy(g_hbm.at[isrc_v.at[j + 1]], buf1, sem1)
                pltpu.sync_copy(buf0, acc_sh.at[idst_v.at[j]], add=True)
                pltpu.make_async_copy(dummy, buf1, sem1).wait()

                @pl.when(j + 2 < half)
                def _():
                    pltpu.async_copy(g_hbm.at[isrc_v.at[j + 2]], buf0, sem0)

                pltpu.sync_copy(buf1, acc_sh.at[idst_v.at[j + 1]], add=True)

        plsc.subcore_barrier()
        pltpu.sync_copy(acc_sh.at[pl.ds(s * RS, RS)],
                        out_hbm.at[c].at[pl.ds(s * RS, RS)])

    return k(g, src2d, dst2d)


def _sc_pair_gather(h, src2d, dst2d):
    @pl.kernel(
        out_type=(jax.ShapeDtypeStruct((EP, D), _f32),
                  jax.ShapeDtypeStruct((EP, D), _f32)),
        mesh=_mesh,
        scratch_types=[
            pltpu.VMEM((NBW, 128), jnp.int32),
            pltpu.VMEM((NBW, 128), jnp.int32),
            pltpu.VMEM((128, D), _f32),
            pltpu.VMEM((128, D), _f32),
            pltpu.VMEM((128, D), _f32),
            pltpu.VMEM((128, D), _f32),
            pltpu.SemaphoreType.DMA,
            pltpu.SemaphoreType.DMA,
            pltpu.SemaphoreType.DMA,
            pltpu.SemaphoreType.DMA,
        ],
    )
    def k(h_hbm, src_hbm, dst_hbm, hs_hbm, hd_hbm, isrc_v, idst_v,
          bs0, bd0, bs1, bd1, ss0, sd0, ss1, sd1):
        c = lax.axis_index("c")
        s = lax.axis_index("s")
        w = c * 16 + s

        pltpu.sync_copy(src_hbm.at[pl.ds(w * NBW, NBW)], isrc_v)
        pltpu.sync_copy(dst_hbm.at[pl.ds(w * NBW, NBW)], idst_v)

        dummy = h_hbm.at[pl.ds(0, 128)]
        pltpu.async_copy(h_hbm.at[isrc_v.at[0]], bs0, ss0)
        pltpu.async_copy(h_hbm.at[idst_v.at[0]], bd0, sd0)

        @pl.loop(0, NBW, step=2)
        def _(j):
            row0 = (w * NBW + j) * 128
            pltpu.make_async_copy(dummy, bs0, ss0).wait()
            pltpu.make_async_copy(dummy, bd0, sd0).wait()
            pltpu.async_copy(h_hbm.at[isrc_v.at[j + 1]], bs1, ss1)
            pltpu.async_copy(h_hbm.at[idst_v.at[j + 1]], bd1, sd1)
            pltpu.sync_copy(bs0, hs_hbm.at[pl.ds(row0, 128)])
            pltpu.sync_copy(bd0, hd_hbm.at[pl.ds(row0, 128)])
            pltpu.make_async_copy(dummy, bs1, ss1).wait()
            pltpu.make_async_copy(dummy, bd1, sd1).wait()

            @pl.when(j + 2 < NBW)
            def _():
                pltpu.async_copy(h_hbm.at[isrc_v.at[j + 2]], bs0, ss0)
                pltpu.async_copy(h_hbm.at[idst_v.at[j + 2]], bd0, sd0)

            pltpu.sync_copy(bs1, hs_hbm.at[pl.ds(row0 + 128, 128)])
            pltpu.sync_copy(bd1, hd_hbm.at[pl.ds(row0 + 128, 128)])

    return k(h, src2d, dst2d)


_R = 2048


def _tc_scale0(xp, degp):
    def body(x_ref, d_ref, o_ref):
        deg = d_ref[0] + d_ref[1] - 1.0
        o_ref[...] = x_ref[...] * lax.rsqrt(deg)

    return pl.pallas_call(
        body,
        grid=(NP // _R,),
        in_specs=[pl.BlockSpec((_R, D), lambda i: (i, 0)),
                  pl.BlockSpec((2, _R, 1), lambda i: (0, i, 0))],
        out_specs=pl.BlockSpec((_R, D), lambda i: (i, 0)),
        out_shape=jax.ShapeDtypeStruct((NP, D), _f32),
    )(xp, degp)


def _tc_combine_mid(p, g, degp):
    def body(p_ref, g_ref, d_ref, o_ref):
        deg = d_ref[0] + d_ref[1] - 1.0
        o_ref[...] = (p_ref[0] + p_ref[1] - g_ref[...]) / deg

    return pl.pallas_call(
        body,
        grid=(NP // _R,),
        in_specs=[pl.BlockSpec((2, _R, D), lambda i: (0, i, 0)),
                  pl.BlockSpec((_R, D), lambda i: (i, 0)),
                  pl.BlockSpec((2, _R, 1), lambda i: (0, i, 0))],
        out_specs=pl.BlockSpec((_R, D), lambda i: (i, 0)),
        out_shape=jax.ShapeDtypeStruct((NP, D), _f32),
    )(p, g, degp)


def _tc_final(p, g, degp, W_sgc, b_sgc):
    def body(p_ref, g_ref, d_ref, w_ref, b_ref, o_ref):
        deg = d_ref[0] + d_ref[1] - 1.0
        hpre = (p_ref[0] + p_ref[1] - g_ref[...]) * lax.rsqrt(deg)
        o_ref[...] = jnp.dot(hpre, w_ref[...],
                             preferred_element_type=_f32) + b_ref[...]

    return pl.pallas_call(
        body,
        grid=(NP // _R,),
        in_specs=[pl.BlockSpec((2, _R, D), lambda i: (0, i, 0)),
                  pl.BlockSpec((_R, D), lambda i: (i, 0)),
                  pl.BlockSpec((2, _R, 1), lambda i: (0, i, 0)),
                  pl.BlockSpec((D, D), lambda i: (0, 0)),
                  pl.BlockSpec((1, D), lambda i: (0, 0))],
        out_specs=pl.BlockSpec((_R, D), lambda i: (i, 0)),
        out_shape=jax.ShapeDtypeStruct((NP, D), _f32),
    )(p, g, degp, W_sgc, b_sgc)


def _tc_mlp(hs, hd, W1, b1, W2, b2, W3, b3):
    def body(hs_ref, hd_ref, w1_ref, b1_ref, w2_ref, b2_ref, w3_ref, b3_ref,
             o_ref):
        z = hs_ref[...] * hd_ref[...]
        z = jnp.maximum(
            jnp.dot(z, w1_ref[...], preferred_element_type=_f32) + b1_ref[...],
            0.0)
        z = jnp.maximum(
            jnp.dot(z, w2_ref[...], preferred_element_type=_f32) + b2_ref[...],
            0.0)
        o_ref[...] = (jnp.dot(z, w3_ref[...], preferred_element_type=_f32)
                      + b3_ref[0, 0])

    return pl.pallas_call(
        body,
        grid=(EP // _R,),
        in_specs=[pl.BlockSpec((_R, D), lambda i: (i, 0)),
                  pl.BlockSpec((_R, D), lambda i: (i, 0)),
                  pl.BlockSpec((D, D), lambda i: (0, 0)),
                  pl.BlockSpec((1, D), lambda i: (0, 0)),
                  pl.BlockSpec((D, D), lambda i: (0, 0)),
                  pl.BlockSpec((1, D), lambda i: (0, 0)),
                  pl.BlockSpec((D, 1), lambda i: (0, 0)),
                  pl.BlockSpec((1, 1), lambda i: (0, 0))],
        out_specs=pl.BlockSpec((_R, 1), lambda i: (i, 0)),
        out_shape=jax.ShapeDtypeStruct((EP, 1), _f32),
    )(hs, hd, W1, b1, W2, b2, W3, b3)


def _pad_idx(row, fill):
    pad = jnp.full((EP - E,), fill, jnp.int32)
    return jnp.concatenate([row, pad]).reshape(EB, 128)


def kernel(x, pos_edge_index, neg_edge_index, W_sgc, b_sgc, W1, b1, W2, b2,
           W3, b3):
    xp = jnp.pad(x, ((0, NP - N), (0, 0)))
    psrc = _pad_idx(pos_edge_index[0], 0)
    pdst = _pad_idx(pos_edge_index[1], N)
    nsrc = _pad_idx(neg_edge_index[0], 0)
    ndst = _pad_idx(neg_edge_index[1], N)

    psrc64 = psrc.reshape(EB64, 64)
    pdst64 = pdst.reshape(EB64, 64)

    degp = _sc_degree(pdst).reshape(2, NP, 1)
    g = _tc_scale0(xp, degp)
    h = None
    for hop in range(3):
        p = _sc_hop(g, psrc64, pdst64)
        if hop < 2:
            g = _tc_combine_mid(p, g, degp)
        else:
            h = _tc_final(p, g, degp, W_sgc, b_sgc.reshape(1, D))

    hs_p, hd_p = _sc_pair_gather(h, psrc, pdst)
    hs_n, hd_n = _sc_pair_gather(h, nsrc, ndst)
    b1r, b2r, b3r = b1.reshape(1, D), b2.reshape(1, D), b3.reshape(1, 1)
    op = _tc_mlp(hs_p, hd_p, W1, b1r, W2, b2r, W3, b3r)
    on = _tc_mlp(hs_n, hd_n, W1, b1r, W2, b2r, W3, b3r)
    return (op[:E], on[:E])


# speedup vs baseline: 2.9054x; 1.0009x over previous
import jax
import jax.numpy as jnp
from jax import lax
from jax.experimental import pallas as pl
from jax.experimental.pallas import tpu as pltpu
from jax.experimental.pallas import tpu_sc as plsc

N = 10000
NP = 10240
D = 128
E = 320000
EP = 327680
EB = EP // 128
NW = 32
NBW = EB // NW
EB64 = EP // 64
NBW64 = EB64 // NW
RS = NP // 16

_mesh = plsc.VectorSubcoreMesh(core_axis_name="c", subcore_axis_name="s")
_f32 = jnp.float32


def _sc_degree(dst2d):
    @pl.kernel(
        out_type=jax.ShapeDtypeStruct((2, NP), _f32),
        mesh=_mesh,
        scratch_types=[
            pltpu.VMEM((RS,), _f32),
            pltpu.VMEM((NBW, 128), jnp.int32),
            pltpu.VMEM_SHARED((NP,), _f32),
        ],
    )
    def k(dst_hbm, out_hbm, ones_v, idx_v, acc_sh):
        c = lax.axis_index("c")
        s = lax.axis_index("s")
        w = c * 16 + s

        @pl.loop(0, RS // 16)
        def _(i):
            ones_v[pl.ds(i * 16, 16)] = jnp.ones((16,), _f32)

        pltpu.sync_copy(ones_v, acc_sh.at[pl.ds(s * RS, RS)])
        pltpu.sync_copy(dst_hbm.at[pl.ds(w * NBW, NBW)], idx_v)
        plsc.subcore_barrier()

        @pl.loop(0, NBW)
        def _(j):
            pltpu.sync_copy(ones_v.at[pl.ds(0, 128)], acc_sh.at[idx_v.at[j]],
                            add=True)

        plsc.subcore_barrier()
        pltpu.sync_copy(acc_sh.at[pl.ds(s * RS, RS)],
                        out_hbm.at[c].at[pl.ds(s * RS, RS)])

    return k(dst2d)


def _sc_hop(g, src2d, dst2d):
    @pl.kernel(
        out_type=jax.ShapeDtypeStruct((2, NP, D), _f32),
        mesh=_mesh,
        scratch_types=[
            pltpu.VMEM((NBW64 // 2, 64), jnp.int32),
            pltpu.VMEM((NBW64 // 2, 64), jnp.int32),
            pltpu.VMEM((64, D), _f32),
            pltpu.VMEM((64, D), _f32),
            pltpu.SemaphoreType.DMA,
            pltpu.SemaphoreType.DMA,
            pltpu.VMEM_SHARED((NP, D), _f32),
        ],
    )
    def k(g_hbm, src_hbm, dst_hbm, out_hbm, isrc_v, idst_v, buf0, buf1,
          sem0, sem1, acc_sh):
        c = lax.axis_index("c")
        s = lax.axis_index("s")
        w = c * 16 + s
        half = NBW64 // 2

        pltpu.sync_copy(g_hbm.at[pl.ds(s * RS, RS)],
                        acc_sh.at[pl.ds(s * RS, RS)])
        plsc.subcore_barrier()

        dummy = g_hbm.at[pl.ds(0, 64)]
        for phase in range(2):
            base = w * NBW64 + phase * half
            pltpu.sync_copy(src_hbm.at[pl.ds(base, half)], isrc_v)
            pltpu.sync_copy(dst_hbm.at[pl.ds(base, half)], idst_v)
            pltpu.async_copy(g_hbm.at[isrc_v.at[0]], buf0, sem0)

            @pl.loop(0, half, step=2)
            def _(j):
                pltpu.make_async_copy(dummy, buf0, sem0).wait()
                pltpu.async_copy(g_hbm.at[isrc_v.at[j + 1]], buf1, sem1)
                pltpu.make_async_copy(dummy, buf1, sem1).wait()

                @pl.when(j + 2 < half)
                def _():
                    pltpu.async_copy(g_hbm.at[isrc_v.at[j + 2]], buf0, sem0)

        plsc.subcore_barrier()
        pltpu.sync_copy(acc_sh.at[pl.ds(s * RS, RS)],
                        out_hbm.at[c].at[pl.ds(s * RS, RS)])

    return k(g, src2d, dst2d)


def _sc_pair_gather(h, src2d, dst2d):
    @pl.kernel(
        out_type=(jax.ShapeDtypeStruct((EP, D), _f32),
                  jax.ShapeDtypeStruct((EP, D), _f32)),
        mesh=_mesh,
        scratch_types=[
            pltpu.VMEM((NBW, 128), jnp.int32),
            pltpu.VMEM((NBW, 128), jnp.int32),
            pltpu.VMEM((128, D), _f32),
            pltpu.VMEM((128, D), _f32),
            pltpu.VMEM((128, D), _f32),
            pltpu.VMEM((128, D), _f32),
            pltpu.SemaphoreType.DMA,
            pltpu.SemaphoreType.DMA,
            pltpu.SemaphoreType.DMA,
            pltpu.SemaphoreType.DMA,
        ],
    )
    def k(h_hbm, src_hbm, dst_hbm, hs_hbm, hd_hbm, isrc_v, idst_v,
          bs0, bd0, bs1, bd1, ss0, sd0, ss1, sd1):
        c = lax.axis_index("c")
        s = lax.axis_index("s")
        w = c * 16 + s

        pltpu.sync_copy(src_hbm.at[pl.ds(w * NBW, NBW)], isrc_v)
        pltpu.sync_copy(dst_hbm.at[pl.ds(w * NBW, NBW)], idst_v)

        dummy = h_hbm.at[pl.ds(0, 128)]
        pltpu.async_copy(h_hbm.at[isrc_v.at[0]], bs0, ss0)
        pltpu.async_copy(h_hbm.at[idst_v.at[0]], bd0, sd0)

        @pl.loop(0, NBW, step=2)
        def _(j):
            row0 = (w * NBW + j) * 128
            pltpu.make_async_copy(dummy, bs0, ss0).wait()
            pltpu.make_async_copy(dummy, bd0, sd0).wait()
            pltpu.async_copy(h_hbm.at[isrc_v.at[j + 1]], bs1, ss1)
            pltpu.async_copy(h_hbm.at[idst_v.at[j + 1]], bd1, sd1)
            pltpu.sync_copy(bs0, hs_hbm.at[pl.ds(row0, 128)])
            pltpu.sync_copy(bd0, hd_hbm.at[pl.ds(row0, 128)])
            pltpu.make_async_copy(dummy, bs1, ss1).wait()
            pltpu.make_async_copy(dummy, bd1, sd1).wait()

            @pl.when(j + 2 < NBW)
            def _():
                pltpu.async_copy(h_hbm.at[isrc_v.at[j + 2]], bs0, ss0)
                pltpu.async_copy(h_hbm.at[idst_v.at[j + 2]], bd0, sd0)

            pltpu.sync_copy(bs1, hs_hbm.at[pl.ds(row0 + 128, 128)])
            pltpu.sync_copy(bd1, hd_hbm.at[pl.ds(row0 + 128, 128)])

    return k(h, src2d, dst2d)


_R = 2048


def _tc_scale0(xp, degp):
    def body(x_ref, d_ref, o_ref):
        deg = d_ref[0] + d_ref[1] - 1.0
        o_ref[...] = x_ref[...] * lax.rsqrt(deg)

    return pl.pallas_call(
        body,
        grid=(NP // _R,),
        in_specs=[pl.BlockSpec((_R, D), lambda i: (i, 0)),
                  pl.BlockSpec((2, _R, 1), lambda i: (0, i, 0))],
        out_specs=pl.BlockSpec((_R, D), lambda i: (i, 0)),
        out_shape=jax.ShapeDtypeStruct((NP, D), _f32),
    )(xp, degp)


def _tc_combine_mid(p, g, degp):
    def body(p_ref, g_ref, d_ref, o_ref):
        deg = d_ref[0] + d_ref[1] - 1.0
        o_ref[...] = (p_ref[0] + p_ref[1] - g_ref[...]) / deg

    return pl.pallas_call(
        body,
        grid=(NP // _R,),
        in_specs=[pl.BlockSpec((2, _R, D), lambda i: (0, i, 0)),
                  pl.BlockSpec((_R, D), lambda i: (i, 0)),
                  pl.BlockSpec((2, _R, 1), lambda i: (0, i, 0))],
        out_specs=pl.BlockSpec((_R, D), lambda i: (i, 0)),
        out_shape=jax.ShapeDtypeStruct((NP, D), _f32),
    )(p, g, degp)


def _tc_final(p, g, degp, W_sgc, b_sgc):
    def body(p_ref, g_ref, d_ref, w_ref, b_ref, o_ref):
        deg = d_ref[0] + d_ref[1] - 1.0
        hpre = (p_ref[0] + p_ref[1] - g_ref[...]) * lax.rsqrt(deg)
        o_ref[...] = jnp.dot(hpre, w_ref[...],
                             preferred_element_type=_f32) + b_ref[...]

    return pl.pallas_call(
        body,
        grid=(NP // _R,),
        in_specs=[pl.BlockSpec((2, _R, D), lambda i: (0, i, 0)),
                  pl.BlockSpec((_R, D), lambda i: (i, 0)),
                  pl.BlockSpec((2, _R, 1), lambda i: (0, i, 0)),
                  pl.BlockSpec((D, D), lambda i: (0, 0)),
                  pl.BlockSpec((1, D), lambda i: (0, 0))],
        out_specs=pl.BlockSpec((_R, D), lambda i: (i, 0)),
        out_shape=jax.ShapeDtypeStruct((NP, D), _f32),
    )(p, g, degp, W_sgc, b_sgc)


def _tc_mlp(hs, hd, W1, b1, W2, b2, W3, b3):
    def body(hs_ref, hd_ref, w1_ref, b1_ref, w2_ref, b2_ref, w3_ref, b3_ref,
             o_ref):
        z = hs_ref[...] * hd_ref[...]
        z = jnp.maximum(
            jnp.dot(z, w1_ref[...], preferred_element_type=_f32) + b1_ref[...],
            0.0)
        z = jnp.maximum(
            jnp.dot(z, w2_ref[...], preferred_element_type=_f32) + b2_ref[...],
            0.0)
        o_ref[...] = (jnp.dot(z, w3_ref[...], preferred_element_type=_f32)
                      + b3_ref[0, 0])

    return pl.pallas_call(
        body,
        grid=(EP // _R,),
        in_specs=[pl.BlockSpec((_R, D), lambda i: (i, 0)),
                  pl.BlockSpec((_R, D), lambda i: (i, 0)),
                  pl.BlockSpec((D, D), lambda i: (0, 0)),
                  pl.BlockSpec((1, D), lambda i: (0, 0)),
                  pl.BlockSpec((D, D), lambda i: (0, 0)),
                  pl.BlockSpec((1, D), lambda i: (0, 0)),
                  pl.BlockSpec((D, 1), lambda i: (0, 0)),
                  pl.BlockSpec((1, 1), lambda i: (0, 0))],
        out_specs=pl.BlockSpec((_R, 1), lambda i: (i, 0)),
        out_shape=jax.ShapeDtypeStruct((EP, 1), _f32),
    )(hs, hd, W1, b1, W2, b2, W3, b3)


def _pad_idx(row, fill):
    pad = jnp.full((EP - E,), fill, jnp.int32)
    return jnp.concatenate([row, pad]).reshape(EB, 128)


def kernel(x, pos_edge_index, neg_edge_index, W_sgc, b_sgc, W1, b1, W2, b2,
           W3, b3):
    xp = jnp.pad(x, ((0, NP - N), (0, 0)))
    psrc = _pad_idx(pos_edge_index[0], 0)
    pdst = _pad_idx(pos_edge_index[1], N)
    nsrc = _pad_idx(neg_edge_index[0], 0)
    ndst = _pad_idx(neg_edge_index[1], N)

    psrc64 = psrc.reshape(EB64, 64)
    pdst64 = pdst.reshape(EB64, 64)

    degp = _sc_degree(pdst).reshape(2, NP, 1)
    g = _tc_scale0(xp, degp)
    h = None
    for hop in range(3):
        p = _sc_hop(g, psrc64, pdst64)
        if hop < 2:
            g = _tc_combine_mid(p, g, degp)
        else:
            h = _tc_final(p, g, degp, W_sgc, b_sgc.reshape(1, D))

    hs_p, hd_p = _sc_pair_gather(h, psrc, pdst)
    hs_n, hd_n = _sc_pair_gather(h, nsrc, ndst)
    b1r, b2r, b3r = b1.reshape(1, D), b2.reshape(1, D), b3.reshape(1, 1)
    op = _tc_mlp(hs_p, hd_p, W1, b1r, W2, b2r, W3, b3r)
    on = _tc_mlp(hs_n, hd_n, W1, b1r, W2, b2r, W3, b3r)
    return (op[:E], on[:E])


# speedup vs baseline: 2.9533x; 1.0165x over previous
import jax
import jax.numpy as jnp
from jax import lax
from jax.experimental import pallas as pl
from jax.experimental.pallas import tpu as pltpu
from jax.experimental.pallas import tpu_sc as plsc

N = 10000
NP = 10240
D = 128
E = 320000
EP = 327680
EB = EP // 128
NW = 32
NBW = EB // NW
EB64 = EP // 64
NBW64 = EB64 // NW
RS = NP // 16

_mesh = plsc.VectorSubcoreMesh(core_axis_name="c", subcore_axis_name="s")
_f32 = jnp.float32


def _sc_degree(dst2d):
    @pl.kernel(
        out_type=jax.ShapeDtypeStruct((2, NP), _f32),
        mesh=_mesh,
        scratch_types=[
            pltpu.VMEM((RS,), _f32),
            pltpu.VMEM((NBW, 128), jnp.int32),
            pltpu.VMEM_SHARED((NP,), _f32),
        ],
    )
    def k(dst_hbm, out_hbm, ones_v, idx_v, acc_sh):
        c = lax.axis_index("c")
        s = lax.axis_index("s")
        w = c * 16 + s

        @pl.loop(0, RS // 16)
        def _(i):
            ones_v[pl.ds(i * 16, 16)] = jnp.ones((16,), _f32)

        pltpu.sync_copy(ones_v, acc_sh.at[pl.ds(s * RS, RS)])
        pltpu.sync_copy(dst_hbm.at[pl.ds(w * NBW, NBW)], idx_v)
        plsc.subcore_barrier()

        @pl.loop(0, NBW)
        def _(j):
            pltpu.sync_copy(ones_v.at[pl.ds(0, 128)], acc_sh.at[idx_v.at[j]],
                            add=True)

        plsc.subcore_barrier()
        pltpu.sync_copy(acc_sh.at[pl.ds(s * RS, RS)],
                        out_hbm.at[c].at[pl.ds(s * RS, RS)])

    return k(dst2d)


def _sc_hop(g, src2d, dst2d):
    @pl.kernel(
        out_type=jax.ShapeDtypeStruct((2, NP, D), _f32),
        mesh=_mesh,
        scratch_types=[
            pltpu.VMEM((NBW64 // 2, 64), jnp.int32),
            pltpu.VMEM((NBW64 // 2, 64), jnp.int32),
            [pltpu.VMEM((64, D), _f32)] * 2,
            [pltpu.SemaphoreType.DMA] * 2,
            [pltpu.SemaphoreType.DMA] * 2,
            pltpu.VMEM_SHARED((NP, D), _f32),
        ],
    )
    def k(g_hbm, src_hbm, dst_hbm, out_hbm, isrc_v, idst_v, bufs, gsem,
          csem, acc_sh):
        c = lax.axis_index("c")
        s = lax.axis_index("s")
        w = c * 16 + s
        half = NBW64 // 2

        pltpu.sync_copy(g_hbm.at[pl.ds(s * RS, RS)],
                        acc_sh.at[pl.ds(s * RS, RS)])
        plsc.subcore_barrier()

        dummy = g_hbm.at[pl.ds(0, 64)]

        def gwait(sl):
            pltpu.make_async_copy(dummy, bufs[sl], gsem[sl]).wait()

        def cwait(sl):
            pltpu.make_async_copy(bufs[sl], acc_sh.at[idst_v.at[0]],
                                  csem[sl]).wait()

        for phase in range(2):
            base = w * NBW64 + phase * half
            pltpu.sync_copy(src_hbm.at[pl.ds(base, half)], isrc_v)
            pltpu.sync_copy(dst_hbm.at[pl.ds(base, half)], idst_v)
            pltpu.async_copy(g_hbm.at[isrc_v.at[0]], bufs[0], gsem[0])
            pltpu.async_copy(g_hbm.at[isrc_v.at[1]], bufs[1], gsem[1])

            @pl.loop(0, half - 2, step=2)
            def _(j):
                gwait(0)
                pltpu.async_copy(bufs[0], acc_sh.at[idst_v.at[j]],
                                 csem[0], add=True)
                gwait(1)
                pltpu.async_copy(bufs[1], acc_sh.at[idst_v.at[j + 1]],
                                 csem[1], add=True)
                cwait(0)
                pltpu.async_copy(g_hbm.at[isrc_v.at[j + 2]], bufs[0],
                                 gsem[0])
                cwait(1)
                pltpu.async_copy(g_hbm.at[isrc_v.at[j + 3]], bufs[1],
                                 gsem[1])

            gwait(0)
            pltpu.async_copy(bufs[0], acc_sh.at[idst_v.at[half - 2]],
                             csem[0], add=True)
            gwait(1)
            pltpu.async_copy(bufs[1], acc_sh.at[idst_v.at[half - 1]],
                             csem[1], add=True)
            cwait(0)
            cwait(1)

        plsc.subcore_barrier()
        pltpu.sync_copy(acc_sh.at[pl.ds(s * RS, RS)],
                        out_hbm.at[c].at[pl.ds(s * RS, RS)])

    return k(g, src2d, dst2d)


def _sc_pair_gather(h, src2d, dst2d):
    @pl.kernel(
        out_type=(jax.ShapeDtypeStruct((EP, D), _f32),
                  jax.ShapeDtypeStruct((EP, D), _f32)),
        mesh=_mesh,
        scratch_types=[
            pltpu.VMEM((NBW, 128), jnp.int32),
            pltpu.VMEM((NBW, 128), jnp.int32),
            pltpu.VMEM((128, D), _f32),
            pltpu.VMEM((128, D), _f32),
            pltpu.VMEM((128, D), _f32),
            pltpu.VMEM((128, D), _f32),
            pltpu.SemaphoreType.DMA,
            pltpu.SemaphoreType.DMA,
            pltpu.SemaphoreType.DMA,
            pltpu.SemaphoreType.DMA,
        ],
    )
    def k(h_hbm, src_hbm, dst_hbm, hs_hbm, hd_hbm, isrc_v, idst_v,
          bs0, bd0, bs1, bd1, ss0, sd0, ss1, sd1):
        c = lax.axis_index("c")
        s = lax.axis_index("s")
        w = c * 16 + s

        pltpu.sync_copy(src_hbm.at[pl.ds(w * NBW, NBW)], isrc_v)
        pltpu.sync_copy(dst_hbm.at[pl.ds(w * NBW, NBW)], idst_v)

        dummy = h_hbm.at[pl.ds(0, 128)]
        pltpu.async_copy(h_hbm.at[isrc_v.at[0]], bs0, ss0)
        pltpu.async_copy(h_hbm.at[idst_v.at[0]], bd0, sd0)

        @pl.loop(0, NBW, step=2)
        def _(j):
            row0 = (w * NBW + j) * 128
            pltpu.make_async_copy(dummy, bs0, ss0).wait()
            pltpu.make_async_copy(dummy, bd0, sd0).wait()
            pltpu.async_copy(h_hbm.at[isrc_v.at[j + 1]], bs1, ss1)
            pltpu.async_copy(h_hbm.at[idst_v.at[j + 1]], bd1, sd1)
            pltpu.sync_copy(bs0, hs_hbm.at[pl.ds(row0, 128)])
            pltpu.sync_copy(bd0, hd_hbm.at[pl.ds(row0, 128)])
            pltpu.make_async_copy(dummy, bs1, ss1).wait()
            pltpu.make_async_copy(dummy, bd1, sd1).wait()

            @pl.when(j + 2 < NBW)
            def _():
                pltpu.async_copy(h_hbm.at[isrc_v.at[j + 2]], bs0, ss0)
                pltpu.async_copy(h_hbm.at[idst_v.at[j + 2]], bd0, sd0)

            pltpu.sync_copy(bs1, hs_hbm.at[pl.ds(row0 + 128, 128)])
            pltpu.sync_copy(bd1, hd_hbm.at[pl.ds(row0 + 128, 128)])

    return k(h, src2d, dst2d)


_R = 2048


def _tc_scale0(xp, degp):
    def body(x_ref, d_ref, o_ref):
        deg = d_ref[0] + d_ref[1] - 1.0
        o_ref[...] = x_ref[...] * lax.rsqrt(deg)

    return pl.pallas_call(
        body,
        grid=(NP // _R,),
        in_specs=[pl.BlockSpec((_R, D), lambda i: (i, 0)),
                  pl.BlockSpec((2, _R, 1), lambda i: (0, i, 0))],
        out_specs=pl.BlockSpec((_R, D), lambda i: (i, 0)),
        out_shape=jax.ShapeDtypeStruct((NP, D), _f32),
    )(xp, degp)


def _tc_combine_mid(p, g, degp):
    def body(p_ref, g_ref, d_ref, o_ref):
        deg = d_ref[0] + d_ref[1] - 1.0
        o_ref[...] = (p_ref[0] + p_ref[1] - g_ref[...]) / deg

    return pl.pallas_call(
        body,
        grid=(NP // _R,),
        in_specs=[pl.BlockSpec((2, _R, D), lambda i: (0, i, 0)),
                  pl.BlockSpec((_R, D), lambda i: (i, 0)),
                  pl.BlockSpec((2, _R, 1), lambda i: (0, i, 0))],
        out_specs=pl.BlockSpec((_R, D), lambda i: (i, 0)),
        out_shape=jax.ShapeDtypeStruct((NP, D), _f32),
    )(p, g, degp)


def _tc_final(p, g, degp, W_sgc, b_sgc):
    def body(p_ref, g_ref, d_ref, w_ref, b_ref, o_ref):
        deg = d_ref[0] + d_ref[1] - 1.0
        hpre = (p_ref[0] + p_ref[1] - g_ref[...]) * lax.rsqrt(deg)
        o_ref[...] = jnp.dot(hpre, w_ref[...],
                             preferred_element_type=_f32) + b_ref[...]

    return pl.pallas_call(
        body,
        grid=(NP // _R,),
        in_specs=[pl.BlockSpec((2, _R, D), lambda i: (0, i, 0)),
                  pl.BlockSpec((_R, D), lambda i: (i, 0)),
                  pl.BlockSpec((2, _R, 1), lambda i: (0, i, 0)),
                  pl.BlockSpec((D, D), lambda i: (0, 0)),
                  pl.BlockSpec((1, D), lambda i: (0, 0))],
        out_specs=pl.BlockSpec((_R, D), lambda i: (i, 0)),
        out_shape=jax.ShapeDtypeStruct((NP, D), _f32),
    )(p, g, degp, W_sgc, b_sgc)


def _tc_mlp(hs, hd, W1, b1, W2, b2, W3, b3):
    def body(hs_ref, hd_ref, w1_ref, b1_ref, w2_ref, b2_ref, w3_ref, b3_ref,
             o_ref):
        z = hs_ref[...] * hd_ref[...]
        z = jnp.maximum(
            jnp.dot(z, w1_ref[...], preferred_element_type=_f32) + b1_ref[...],
            0.0)
        z = jnp.maximum(
            jnp.dot(z, w2_ref[...], preferred_element_type=_f32) + b2_ref[...],
            0.0)
        o_ref[...] = (jnp.dot(z, w3_ref[...], preferred_element_type=_f32)
                      + b3_ref[0, 0])

    return pl.pallas_call(
        body,
        grid=(EP // _R,),
        in_specs=[pl.BlockSpec((_R, D), lambda i: (i, 0)),
                  pl.BlockSpec((_R, D), lambda i: (i, 0)),
                  pl.BlockSpec((D, D), lambda i: (0, 0)),
                  pl.BlockSpec((1, D), lambda i: (0, 0)),
                  pl.BlockSpec((D, D), lambda i: (0, 0)),
                  pl.BlockSpec((1, D), lambda i: (0, 0)),
                  pl.BlockSpec((D, 1), lambda i: (0, 0)),
                  pl.BlockSpec((1, 1), lambda i: (0, 0))],
        out_specs=pl.BlockSpec((_R, 1), lambda i: (i, 0)),
        out_shape=jax.ShapeDtypeStruct((EP, 1), _f32),
    )(hs, hd, W1, b1, W2, b2, W3, b3)


def _pad_idx(row, fill):
    pad = jnp.full((EP - E,), fill, jnp.int32)
    return jnp.concatenate([row, pad]).reshape(EB, 128)


def kernel(x, pos_edge_index, neg_edge_index, W_sgc, b_sgc, W1, b1, W2, b2,
           W3, b3):
    xp = jnp.pad(x, ((0, NP - N), (0, 0)))
    psrc = _pad_idx(pos_edge_index[0], 0)
    pdst = _pad_idx(pos_edge_index[1], N)
    nsrc = _pad_idx(neg_edge_index[0], 0)
    ndst = _pad_idx(neg_edge_index[1], N)

    psrc64 = psrc.reshape(EB64, 64)
    pdst64 = pdst.reshape(EB64, 64)

    degp = _sc_degree(pdst).reshape(2, NP, 1)
    g = _tc_scale0(xp, degp)
    h = None
    for hop in range(3):
        p = _sc_hop(g, psrc64, pdst64)
        if hop < 2:
            g = _tc_combine_mid(p, g, degp)
        else:
            h = _tc_final(p, g, degp, W_sgc, b_sgc.reshape(1, D))

    hs_p, hd_p = _sc_pair_gather(h, psrc, pdst)
    hs_n, hd_n = _sc_pair_gather(h, nsrc, ndst)
    b1r, b2r, b3r = b1.reshape(1, D), b2.reshape(1, D), b3.reshape(1, 1)
    op = _tc_mlp(hs_p, hd_p, W1, b1r, W2, b2r, W3, b3r)
    on = _tc_mlp(hs_n, hd_n, W1, b1r, W2, b2r, W3, b3r)
    return (op[:E], on[:E])


# speedup vs baseline: 3.0394x; 1.0291x over previous
import jax
import jax.numpy as jnp
from jax import lax
from jax.experimental import pallas as pl
from jax.experimental.pallas import tpu as pltpu
from jax.experimental.pallas import tpu_sc as plsc

N = 10000
NP = 10240
D = 128
E = 320000
EP = 327680
EB = EP // 128
NW = 32
NBW = EB // NW
EB64 = EP // 64
NBW64 = EB64 // NW
RS = NP // 16

_mesh = plsc.VectorSubcoreMesh(core_axis_name="c", subcore_axis_name="s")
_f32 = jnp.float32


def _sc_degree(dst2d):
    @pl.kernel(
        out_type=jax.ShapeDtypeStruct((2, NP), _f32),
        mesh=_mesh,
        scratch_types=[
            pltpu.VMEM((RS,), _f32),
            pltpu.VMEM((NBW, 128), jnp.int32),
            pltpu.VMEM_SHARED((NP,), _f32),
        ],
    )
    def k(dst_hbm, out_hbm, ones_v, idx_v, acc_sh):
        c = lax.axis_index("c")
        s = lax.axis_index("s")
        w = c * 16 + s

        @pl.loop(0, RS // 16)
        def _(i):
            ones_v[pl.ds(i * 16, 16)] = jnp.ones((16,), _f32)

        pltpu.sync_copy(ones_v, acc_sh.at[pl.ds(s * RS, RS)])
        pltpu.sync_copy(dst_hbm.at[pl.ds(w * NBW, NBW)], idx_v)
        plsc.subcore_barrier()

        @pl.loop(0, NBW)
        def _(j):
            pltpu.sync_copy(ones_v.at[pl.ds(0, 128)], acc_sh.at[idx_v.at[j]],
                            add=True)

        plsc.subcore_barrier()
        pltpu.sync_copy(acc_sh.at[pl.ds(s * RS, RS)],
                        out_hbm.at[c].at[pl.ds(s * RS, RS)])

    return k(dst2d)


def _sc_hop(g, src2d, dst2d):
    @pl.kernel(
        out_type=jax.ShapeDtypeStruct((2, NP, D), _f32),
        mesh=_mesh,
        scratch_types=[
            pltpu.VMEM((NBW64 // 2, 64), jnp.int32),
            pltpu.VMEM((NBW64 // 2, 64), jnp.int32),
            [pltpu.VMEM((64, D), _f32)] * 2,
            [pltpu.SemaphoreType.DMA] * 2,
            [pltpu.SemaphoreType.DMA] * 2,
            pltpu.VMEM_SHARED((NP, D), _f32),
        ],
    )
    def k(g_hbm, src_hbm, dst_hbm, out_hbm, isrc_v, idst_v, bufs, gsem,
          csem, acc_sh):
        c = lax.axis_index("c")
        s = lax.axis_index("s")
        w = c * 16 + s
        half = NBW64 // 2

        pltpu.sync_copy(g_hbm.at[pl.ds(s * RS, RS)],
                        acc_sh.at[pl.ds(s * RS, RS)])
        plsc.subcore_barrier()

        dummy = g_hbm.at[pl.ds(0, 64)]

        def gwait(sl):
            pltpu.make_async_copy(dummy, bufs[sl], gsem[sl]).wait()

        def cwait(sl):
            pltpu.make_async_copy(bufs[sl], acc_sh.at[idst_v.at[0]],
                                  csem[sl]).wait()

        for phase in range(2):
            base = w * NBW64 + phase * half
            pltpu.sync_copy(src_hbm.at[pl.ds(base, half)], isrc_v)
            pltpu.sync_copy(dst_hbm.at[pl.ds(base, half)], idst_v)
            pltpu.async_copy(g_hbm.at[isrc_v.at[0]], bufs[0], gsem[0])
            pltpu.async_copy(g_hbm.at[isrc_v.at[1]], bufs[1], gsem[1])

            @pl.loop(0, half - 2, step=2)
            def _(j):
                gwait(0)
                pltpu.async_copy(bufs[0], acc_sh.at[idst_v.at[j]],
                                 csem[0], add=True)
                gwait(1)
                pltpu.async_copy(bufs[1], acc_sh.at[idst_v.at[j + 1]],
                                 csem[1], add=True)
                cwait(0)
                pltpu.async_copy(g_hbm.at[isrc_v.at[j + 2]], bufs[0],
                                 gsem[0])
                cwait(1)
                pltpu.async_copy(g_hbm.at[isrc_v.at[j + 3]], bufs[1],
                                 gsem[1])

            gwait(0)
            pltpu.async_copy(bufs[0], acc_sh.at[idst_v.at[half - 2]],
                             csem[0], add=True)
            gwait(1)
            pltpu.async_copy(bufs[1], acc_sh.at[idst_v.at[half - 1]],
                             csem[1], add=True)
            cwait(0)
            cwait(1)

        plsc.subcore_barrier()
        pltpu.sync_copy(acc_sh.at[pl.ds(s * RS, RS)],
                        out_hbm.at[c].at[pl.ds(s * RS, RS)])

    return k(g, src2d, dst2d)


def _sc_pair_gather(h, src2d, dst2d):
    @pl.kernel(
        out_type=(jax.ShapeDtypeStruct((EP, D), _f32),
                  jax.ShapeDtypeStruct((EP, D), _f32)),
        mesh=_mesh,
        scratch_types=[
            pltpu.VMEM((NBW // 2, 128), jnp.int32),
            pltpu.VMEM((NBW // 2, 128), jnp.int32),
            [pltpu.VMEM((128, D), _f32)] * 3,
            [pltpu.VMEM((128, D), _f32)] * 3,
            [pltpu.SemaphoreType.DMA] * 3,
            [pltpu.SemaphoreType.DMA] * 3,
        ],
    )
    def k(h_hbm, src_hbm, dst_hbm, hs_hbm, hd_hbm, isrc_v, idst_v,
          bs, bd, gsem, wsem):
        c = lax.axis_index("c")
        s = lax.axis_index("s")
        w = c * 16 + s
        q = NBW // 2

        dummy = h_hbm.at[pl.ds(0, 128)]

        def gwait(sl):
            pltpu.make_async_copy(dummy, bs[sl], gsem[sl]).wait()
            pltpu.make_async_copy(dummy, bd[sl], gsem[sl]).wait()

        def wwait(sl):
            pltpu.make_async_copy(bs[sl], hs_hbm.at[pl.ds(0, 128)],
                                  wsem[sl]).wait()
            pltpu.make_async_copy(bd[sl], hd_hbm.at[pl.ds(0, 128)],
                                  wsem[sl]).wait()

        def gstart(sl, j):
            pltpu.async_copy(h_hbm.at[isrc_v.at[j]], bs[sl], gsem[sl])
            pltpu.async_copy(h_hbm.at[idst_v.at[j]], bd[sl], gsem[sl])

        def wstart(sl, row0):
            pltpu.async_copy(bs[sl], hs_hbm.at[pl.ds(row0, 128)], wsem[sl])
            pltpu.async_copy(bd[sl], hd_hbm.at[pl.ds(row0, 128)], wsem[sl])

        for phase in range(2):
            base = w * NBW + phase * q
            pltpu.sync_copy(src_hbm.at[pl.ds(base, q)], isrc_v)
            pltpu.sync_copy(dst_hbm.at[pl.ds(base, q)], idst_v)
            gstart(0, 0)
            gstart(1, 1)
            gwait(0)
            wstart(0, base * 128)
            gstart(2, 2)
            gwait(1)
            wstart(1, (base + 1) * 128)
            wwait(0)
            gstart(0, 3)

            nfull = (q - 8) // 3
            stop = 2 + 3 * nfull

            @pl.loop(2, stop, step=3)
            def _(j):
                for k3, sl in ((0, 2), (1, 0), (2, 1)):
                    jj = j + k3
                    s2 = (sl + 2) % 3
                    gwait(sl)
                    wstart(sl, (base + jj) * 128)
                    wwait(s2)
                    gstart(s2, jj + 2)

            for j0 in range(stop, q - 2):
                sl = j0 % 3
                s2 = (sl + 2) % 3
                gwait(sl)
                wstart(sl, (base + j0) * 128)
                wwait(s2)
                gstart(s2, j0 + 2)
            for j0 in (q - 2, q - 1):
                sl = j0 % 3
                gwait(sl)
                wstart(sl, (base + j0) * 128)
            for sl in range(3):
                wwait(sl)

    return k(h, src2d, dst2d)


_R = 2048


def _tc_scale0(xp, degp):
    def body(x_ref, d_ref, o_ref):
        deg = d_ref[0] + d_ref[1] - 1.0
        o_ref[...] = x_ref[...] * lax.rsqrt(deg)

    return pl.pallas_call(
        body,
        grid=(NP // _R,),
        in_specs=[pl.BlockSpec((_R, D), lambda i: (i, 0)),
                  pl.BlockSpec((2, _R, 1), lambda i: (0, i, 0))],
        out_specs=pl.BlockSpec((_R, D), lambda i: (i, 0)),
        out_shape=jax.ShapeDtypeStruct((NP, D), _f32),
    )(xp, degp)


def _tc_combine_mid(p, g, degp):
    def body(p_ref, g_ref, d_ref, o_ref):
        deg = d_ref[0] + d_ref[1] - 1.0
        o_ref[...] = (p_ref[0] + p_ref[1] - g_ref[...]) / deg

    return pl.pallas_call(
        body,
        grid=(NP // _R,),
        in_specs=[pl.BlockSpec((2, _R, D), lambda i: (0, i, 0)),
                  pl.BlockSpec((_R, D), lambda i: (i, 0)),
                  pl.BlockSpec((2, _R, 1), lambda i: (0, i, 0))],
        out_specs=pl.BlockSpec((_R, D), lambda i: (i, 0)),
        out_shape=jax.ShapeDtypeStruct((NP, D), _f32),
    )(p, g, degp)


def _tc_final(p, g, degp, W_sgc, b_sgc):
    def body(p_ref, g_ref, d_ref, w_ref, b_ref, o_ref):
        deg = d_ref[0] + d_ref[1] - 1.0
        hpre = (p_ref[0] + p_ref[1] - g_ref[...]) * lax.rsqrt(deg)
        o_ref[...] = jnp.dot(hpre, w_ref[...],
                             preferred_element_type=_f32) + b_ref[...]

    return pl.pallas_call(
        body,
        grid=(NP // _R,),
        in_specs=[pl.BlockSpec((2, _R, D), lambda i: (0, i, 0)),
                  pl.BlockSpec((_R, D), lambda i: (i, 0)),
                  pl.BlockSpec((2, _R, 1), lambda i: (0, i, 0)),
                  pl.BlockSpec((D, D), lambda i: (0, 0)),
                  pl.BlockSpec((1, D), lambda i: (0, 0))],
        out_specs=pl.BlockSpec((_R, D), lambda i: (i, 0)),
        out_shape=jax.ShapeDtypeStruct((NP, D), _f32),
    )(p, g, degp, W_sgc, b_sgc)


def _tc_mlp(hs, hd, W1, b1, W2, b2, W3, b3):
    def body(hs_ref, hd_ref, w1_ref, b1_ref, w2_ref, b2_ref, w3_ref, b3_ref,
             o_ref):
        z = hs_ref[...] * hd_ref[...]
        z = jnp.maximum(
            jnp.dot(z, w1_ref[...], preferred_element_type=_f32) + b1_ref[...],
            0.0)
        z = jnp.maximum(
            jnp.dot(z, w2_ref[...], preferred_element_type=_f32) + b2_ref[...],
            0.0)
        o_ref[...] = (jnp.dot(z, w3_ref[...], preferred_element_type=_f32)
                      + b3_ref[0, 0])

    return pl.pallas_call(
        body,
        grid=(-(-E // _R),),
        in_specs=[pl.BlockSpec((_R, D), lambda i: (i, 0)),
                  pl.BlockSpec((_R, D), lambda i: (i, 0)),
                  pl.BlockSpec((D, D), lambda i: (0, 0)),
                  pl.BlockSpec((1, D), lambda i: (0, 0)),
                  pl.BlockSpec((D, D), lambda i: (0, 0)),
                  pl.BlockSpec((1, D), lambda i: (0, 0)),
                  pl.BlockSpec((D, 1), lambda i: (0, 0)),
                  pl.BlockSpec((1, 1), lambda i: (0, 0))],
        out_specs=pl.BlockSpec((_R, 1), lambda i: (i, 0)),
        out_shape=jax.ShapeDtypeStruct((E, 1), _f32),
    )(hs, hd, W1, b1, W2, b2, W3, b3)


def _pad_idx(row, fill):
    pad = jnp.full((EP - E,), fill, jnp.int32)
    return jnp.concatenate([row, pad]).reshape(EB, 128)


def kernel(x, pos_edge_index, neg_edge_index, W_sgc, b_sgc, W1, b1, W2, b2,
           W3, b3):
    xp = jnp.pad(x, ((0, NP - N), (0, 0)))
    psrc = _pad_idx(pos_edge_index[0], 0)
    pdst = _pad_idx(pos_edge_index[1], N)
    nsrc = _pad_idx(neg_edge_index[0], 0)
    ndst = _pad_idx(neg_edge_index[1], N)

    psrc64 = psrc.reshape(EB64, 64)
    pdst64 = pdst.reshape(EB64, 64)

    degp = _sc_degree(pdst).reshape(2, NP, 1)
    g = _tc_scale0(xp, degp)
    h = None
    for hop in range(3):
        p = _sc_hop(g, psrc64, pdst64)
        if hop < 2:
            g = _tc_combine_mid(p, g, degp)
        else:
            h = _tc_final(p, g, degp, W_sgc, b_sgc.reshape(1, D))

    hs_p, hd_p = _sc_pair_gather(h, psrc, pdst)
    hs_n, hd_n = _sc_pair_gather(h, nsrc, ndst)
    b1r, b2r, b3r = b1.reshape(1, D), b2.reshape(1, D), b3.reshape(1, 1)
    op = _tc_mlp(hs_p, hd_p, W1, b1r, W2, b2r, W3, b3r)
    on = _tc_mlp(hs_n, hd_n, W1, b1r, W2, b2r, W3, b3r)
    return (op, on)


# speedup vs baseline: 3.1445x; 1.0346x over previous
import jax
import jax.numpy as jnp
from jax import lax
from jax.experimental import pallas as pl
from jax.experimental.pallas import tpu as pltpu
from jax.experimental.pallas import tpu_sc as plsc

N = 10000
NP = 10240
D = 128
E = 320000
EP = 327680
EB = EP // 128
NW = 32
NBW = EB // NW
EB64 = EP // 64
NBW64 = EB64 // NW
RS = NP // 16

HFAST = 0
HF0 = 224
HF1 = 96
PF0 = 112
PF1 = 48
PQ = 16

_mesh = plsc.VectorSubcoreMesh(core_axis_name="c", subcore_axis_name="s")
_f32 = jnp.float32


def _sc_degree(dst2d):
    @pl.kernel(
        out_type=jax.ShapeDtypeStruct((2, NP), _f32),
        mesh=_mesh,
        scratch_types=[
            pltpu.VMEM((RS,), _f32),
            pltpu.VMEM((NBW, 128), jnp.int32),
            pltpu.VMEM_SHARED((NP,), _f32),
        ],
    )
    def k(dst_hbm, out_hbm, ones_v, idx_v, acc_sh):
        c = lax.axis_index("c")
        s = lax.axis_index("s")
        w = c * 16 + s

        @pl.loop(0, RS // 16)
        def _(i):
            ones_v[pl.ds(i * 16, 16)] = jnp.ones((16,), _f32)

        pltpu.sync_copy(ones_v, acc_sh.at[pl.ds(s * RS, RS)])
        pltpu.sync_copy(dst_hbm.at[pl.ds(w * NBW, NBW)], idx_v)
        plsc.subcore_barrier()

        @pl.loop(0, NBW)
        def _(j):
            pltpu.sync_copy(ones_v.at[pl.ds(0, 128)], acc_sh.at[idx_v.at[j]],
                            add=True)

        plsc.subcore_barrier()
        pltpu.sync_copy(acc_sh.at[pl.ds(s * RS, RS)],
                        out_hbm.at[c].at[pl.ds(s * RS, RS)])

    return k(dst2d)


def _sc_hop(g, src2d, dst2d):
    @pl.kernel(
        out_type=jax.ShapeDtypeStruct((2, NP, D), _f32),
        mesh=_mesh,
        scratch_types=[
            pltpu.VMEM((HF0 // 4, 64), jnp.int32),
            pltpu.VMEM((HF0 // 4, 64), jnp.int32),
            [pltpu.VMEM((64, D), _f32)] * 2,
            [pltpu.SemaphoreType.DMA] * 2,
            [pltpu.SemaphoreType.DMA] * 2,
            pltpu.VMEM_SHARED((NP, D), _f32),
        ],
    )
    def k(g_hbm, src_hbm, dst_hbm, out_hbm, isrc_v, idst_v, bufs, gsem,
          csem, acc_sh):
        c = lax.axis_index("c")
        s = lax.axis_index("s")

        pltpu.sync_copy(g_hbm.at[pl.ds(s * RS, RS)],
                        acc_sh.at[pl.ds(s * RS, RS)])
        plsc.subcore_barrier()

        dummy = g_hbm.at[pl.ds(0, 64)]

        def gwait(sl):
            pltpu.make_async_copy(dummy, bufs[sl], gsem[sl]).wait()

        def cwait(sl):
            pltpu.make_async_copy(bufs[sl], acc_sh.at[idst_v.at[0]],
                                  csem[sl]).wait()

        def run(base_blocks, nphase, q):
            for phase in range(nphase):
                base = base_blocks + phase * q
                pltpu.sync_copy(src_hbm.at[pl.ds(base, q)],
                                isrc_v.at[pl.ds(0, q)])
                pltpu.sync_copy(dst_hbm.at[pl.ds(base, q)],
                                idst_v.at[pl.ds(0, q)])
                pltpu.async_copy(g_hbm.at[isrc_v.at[0]], bufs[0], gsem[0])
                pltpu.async_copy(g_hbm.at[isrc_v.at[1]], bufs[1], gsem[1])

                @pl.loop(0, q - 2, step=2)
                def _(j):
                    gwait(0)
                    pltpu.async_copy(bufs[0], acc_sh.at[idst_v.at[j]],
                                     csem[0], add=True)
                    gwait(1)
                    pltpu.async_copy(bufs[1], acc_sh.at[idst_v.at[j + 1]],
                                     csem[1], add=True)
                    cwait(0)
                    pltpu.async_copy(g_hbm.at[isrc_v.at[j + 2]], bufs[0],
                                     gsem[0])
                    cwait(1)
                    pltpu.async_copy(g_hbm.at[isrc_v.at[j + 3]], bufs[1],
                                     gsem[1])

                gwait(0)
                pltpu.async_copy(bufs[0], acc_sh.at[idst_v.at[q - 2]],
                                 csem[0], add=True)
                gwait(1)
                pltpu.async_copy(bufs[1], acc_sh.at[idst_v.at[q - 1]],
                                 csem[1], add=True)
                cwait(0)
                cwait(1)

        @pl.when(c == HFAST)
        def _():
            run(s * HF0, 4, HF0 // 4)

        @pl.when(c == 1 - HFAST)
        def _():
            run(16 * HF0 + s * HF1, 2, HF1 // 2)

        plsc.subcore_barrier()
        pltpu.sync_copy(acc_sh.at[pl.ds(s * RS, RS)],
                        out_hbm.at[c].at[pl.ds(s * RS, RS)])

    return k(g, src2d, dst2d)


def _sc_pair_gather(h, src2d, dst2d):
    @pl.kernel(
        out_type=(jax.ShapeDtypeStruct((EP, D), _f32),
                  jax.ShapeDtypeStruct((EP, D), _f32)),
        mesh=_mesh,
        scratch_types=[
            pltpu.VMEM((PQ, 128), jnp.int32),
            pltpu.VMEM((PQ, 128), jnp.int32),
            [pltpu.VMEM((128, D), _f32)] * 3,
            [pltpu.VMEM((128, D), _f32)] * 3,
            [pltpu.SemaphoreType.DMA] * 3,
            [pltpu.SemaphoreType.DMA] * 3,
        ],
    )
    def k(h_hbm, src_hbm, dst_hbm, hs_hbm, hd_hbm, isrc_v, idst_v,
          bs, bd, gsem, wsem):
        c = lax.axis_index("c")
        s = lax.axis_index("s")
        q = PQ

        dummy = h_hbm.at[pl.ds(0, 128)]

        def gwait(sl):
            pltpu.make_async_copy(dummy, bs[sl], gsem[sl]).wait()
            pltpu.make_async_copy(dummy, bd[sl], gsem[sl]).wait()

        def wwait(sl):
            pltpu.make_async_copy(bs[sl], hs_hbm.at[pl.ds(0, 128)],
                                  wsem[sl]).wait()
            pltpu.make_async_copy(bd[sl], hd_hbm.at[pl.ds(0, 128)],
                                  wsem[sl]).wait()

        def gstart(sl, j):
            pltpu.async_copy(h_hbm.at[isrc_v.at[j]], bs[sl], gsem[sl])
            pltpu.async_copy(h_hbm.at[idst_v.at[j]], bd[sl], gsem[sl])

        def wstart(sl, row0):
            pltpu.async_copy(bs[sl], hs_hbm.at[pl.ds(row0, 128)], wsem[sl])
            pltpu.async_copy(bd[sl], hd_hbm.at[pl.ds(row0, 128)], wsem[sl])

        def run(base_blocks, nphase):
          for phase in range(nphase):
            base = base_blocks + phase * q
            pltpu.sync_copy(src_hbm.at[pl.ds(base, q)], isrc_v)
            pltpu.sync_copy(dst_hbm.at[pl.ds(base, q)], idst_v)
            gstart(0, 0)
            gstart(1, 1)
            gwait(0)
            wstart(0, base * 128)
            gstart(2, 2)
            gwait(1)
            wstart(1, (base + 1) * 128)
            wwait(0)
            gstart(0, 3)

            nfull = (q - 8) // 3
            stop = 2 + 3 * nfull

            @pl.loop(2, stop, step=3)
            def _(j):
                for k3, sl in ((0, 2), (1, 0), (2, 1)):
                    jj = j + k3
                    s2 = (sl + 2) % 3
                    gwait(sl)
                    wstart(sl, (base + jj) * 128)
                    wwait(s2)
                    gstart(s2, jj + 2)

            for j0 in range(stop, q - 2):
                sl = j0 % 3
                s2 = (sl + 2) % 3
                gwait(sl)
                wstart(sl, (base + j0) * 128)
                wwait(s2)
                gstart(s2, j0 + 2)
            for j0 in (q - 2, q - 1):
                sl = j0 % 3
                gwait(sl)
                wstart(sl, (base + j0) * 128)
            for sl in range(3):
                wwait(sl)

        @pl.when(c == HFAST)
        def _():
            run(s * PF0, PF0 // PQ)

        @pl.when(c == 1 - HFAST)
        def _():
            run(16 * PF0 + s * PF1, PF1 // PQ)

    return k(h, src2d, dst2d)


_R = 2048


def _tc_scale0(xp, degp):
    def body(x_ref, d_ref, o_ref):
        deg = d_ref[0] + d_ref[1] - 1.0
        o_ref[...] = x_ref[...] * lax.rsqrt(deg)

    return pl.pallas_call(
        body,
        grid=(NP // _R,),
        in_specs=[pl.BlockSpec((_R, D), lambda i: (i, 0)),
                  pl.BlockSpec((2, _R, 1), lambda i: (0, i, 0))],
        out_specs=pl.BlockSpec((_R, D), lambda i: (i, 0)),
        out_shape=jax.ShapeDtypeStruct((NP, D), _f32),
    )(xp, degp)


def _tc_combine_mid(p, g, degp):
    def body(p_ref, g_ref, d_ref, o_ref):
        deg = d_ref[0] + d_ref[1] - 1.0
        o_ref[...] = (p_ref[0] + p_ref[1] - g_ref[...]) / deg

    return pl.pallas_call(
        body,
        grid=(NP // _R,),
        in_specs=[pl.BlockSpec((2, _R, D), lambda i: (0, i, 0)),
                  pl.BlockSpec((_R, D), lambda i: (i, 0)),
                  pl.BlockSpec((2, _R, 1), lambda i: (0, i, 0))],
        out_specs=pl.BlockSpec((_R, D), lambda i: (i, 0)),
        out_shape=jax.ShapeDtypeStruct((NP, D), _f32),
    )(p, g, degp)


def _tc_final(p, g, degp, W_sgc, b_sgc):
    def body(p_ref, g_ref, d_ref, w_ref, b_ref, o_ref):
        deg = d_ref[0] + d_ref[1] - 1.0
        hpre = (p_ref[0] + p_ref[1] - g_ref[...]) * lax.rsqrt(deg)
        o_ref[...] = jnp.dot(hpre, w_ref[...],
                             preferred_element_type=_f32) + b_ref[...]

    return pl.pallas_call(
        body,
        grid=(NP // _R,),
        in_specs=[pl.BlockSpec((2, _R, D), lambda i: (0, i, 0)),
                  pl.BlockSpec((_R, D), lambda i: (i, 0)),
                  pl.BlockSpec((2, _R, 1), lambda i: (0, i, 0)),
                  pl.BlockSpec((D, D), lambda i: (0, 0)),
                  pl.BlockSpec((1, D), lambda i: (0, 0))],
        out_specs=pl.BlockSpec((_R, D), lambda i: (i, 0)),
        out_shape=jax.ShapeDtypeStruct((NP, D), _f32),
    )(p, g, degp, W_sgc, b_sgc)


def _tc_mlp(hs, hd, W1, b1, W2, b2, W3, b3):
    def body(hs_ref, hd_ref, w1_ref, b1_ref, w2_ref, b2_ref, w3_ref, b3_ref,
             o_ref):
        z = hs_ref[...] * hd_ref[...]
        z = jnp.maximum(
            jnp.dot(z, w1_ref[...], preferred_element_type=_f32) + b1_ref[...],
            0.0)
        z = jnp.maximum(
            jnp.dot(z, w2_ref[...], preferred_element_type=_f32) + b2_ref[...],
            0.0)
        o_ref[...] = (jnp.dot(z, w3_ref[...], preferred_element_type=_f32)
                      + b3_ref[0, 0])

    return pl.pallas_call(
        body,
        grid=(-(-E // _R),),
        in_specs=[pl.BlockSpec((_R, D), lambda i: (i, 0)),
                  pl.BlockSpec((_R, D), lambda i: (i, 0)),
                  pl.BlockSpec((D, D), lambda i: (0, 0)),
                  pl.BlockSpec((1, D), lambda i: (0, 0)),
                  pl.BlockSpec((D, D), lambda i: (0, 0)),
                  pl.BlockSpec((1, D), lambda i: (0, 0)),
                  pl.BlockSpec((D, 1), lambda i: (0, 0)),
                  pl.BlockSpec((1, 1), lambda i: (0, 0))],
        out_specs=pl.BlockSpec((_R, 1), lambda i: (i, 0)),
        out_shape=jax.ShapeDtypeStruct((E, 1), _f32),
    )(hs, hd, W1, b1, W2, b2, W3, b3)


def _pad_idx(row, fill):
    pad = jnp.full((EP - E,), fill, jnp.int32)
    return jnp.concatenate([row, pad]).reshape(EB, 128)


def kernel(x, pos_edge_index, neg_edge_index, W_sgc, b_sgc, W1, b1, W2, b2,
           W3, b3):
    xp = jnp.pad(x, ((0, NP - N), (0, 0)))
    psrc = _pad_idx(pos_edge_index[0], 0)
    pdst = _pad_idx(pos_edge_index[1], N)
    nsrc = _pad_idx(neg_edge_index[0], 0)
    ndst = _pad_idx(neg_edge_index[1], N)

    psrc64 = psrc.reshape(EB64, 64)
    pdst64 = pdst.reshape(EB64, 64)

    degp = _sc_degree(pdst).reshape(2, NP, 1)
    g = _tc_scale0(xp, degp)
    h = None
    for hop in range(3):
        p = _sc_hop(g, psrc64, pdst64)
        if hop < 2:
            g = _tc_combine_mid(p, g, degp)
        else:
            h = _tc_final(p, g, degp, W_sgc, b_sgc.reshape(1, D))

    hs_p, hd_p = _sc_pair_gather(h, psrc, pdst)
    hs_n, hd_n = _sc_pair_gather(h, nsrc, ndst)
    b1r, b2r, b3r = b1.reshape(1, D), b2.reshape(1, D), b3.reshape(1, 1)
    op = _tc_mlp(hs_p, hd_p, W1, b1r, W2, b2r, W3, b3r)
    on = _tc_mlp(hs_n, hd_n, W1, b1r, W2, b2r, W3, b3r)
    return (op, on)


# speedup vs baseline: 3.1891x; 1.0142x over previous
import jax
import jax.numpy as jnp
from jax import lax
from jax.experimental import pallas as pl
from jax.experimental.pallas import tpu as pltpu
from jax.experimental.pallas import tpu_sc as plsc

N = 10000
NP = 10240
D = 128
E = 320000
EP = 327680
EB = EP // 128
NW = 32
NBW = EB // NW
EB64 = EP // 64
NBW64 = EB64 // NW
RS = NP // 16

HFAST = 0
HF0 = 256
HF1 = 64
PF0 = 112
PF1 = 48
PQ = 16

_mesh = plsc.VectorSubcoreMesh(core_axis_name="c", subcore_axis_name="s")
_f32 = jnp.float32


def _sc_degree(dst2d):
    @pl.kernel(
        out_type=jax.ShapeDtypeStruct((2, NP), _f32),
        mesh=_mesh,
        scratch_types=[
            pltpu.VMEM((RS,), _f32),
            pltpu.VMEM((NBW, 128), jnp.int32),
            pltpu.VMEM_SHARED((NP,), _f32),
        ],
    )
    def k(dst_hbm, out_hbm, ones_v, idx_v, acc_sh):
        c = lax.axis_index("c")
        s = lax.axis_index("s")
        w = c * 16 + s

        @pl.loop(0, RS // 16)
        def _(i):
            ones_v[pl.ds(i * 16, 16)] = jnp.ones((16,), _f32)

        pltpu.sync_copy(ones_v, acc_sh.at[pl.ds(s * RS, RS)])
        pltpu.sync_copy(dst_hbm.at[pl.ds(w * NBW, NBW)], idx_v)
        plsc.subcore_barrier()

        @pl.loop(0, NBW)
        def _(j):
            pltpu.sync_copy(ones_v.at[pl.ds(0, 128)], acc_sh.at[idx_v.at[j]],
                            add=True)

        plsc.subcore_barrier()
        pltpu.sync_copy(acc_sh.at[pl.ds(s * RS, RS)],
                        out_hbm.at[c].at[pl.ds(s * RS, RS)])

    return k(dst2d)


def _sc_hop(g, src2d, dst2d):
    @pl.kernel(
        out_type=jax.ShapeDtypeStruct((2, NP, D), _f32),
        mesh=_mesh,
        scratch_types=[
            pltpu.VMEM((HF0 // 4, 64), jnp.int32),
            pltpu.VMEM((HF0 // 4, 64), jnp.int32),
            [pltpu.VMEM((64, D), _f32)] * 2,
            [pltpu.SemaphoreType.DMA] * 2,
            [pltpu.SemaphoreType.DMA] * 2,
            pltpu.VMEM_SHARED((NP, D), _f32),
        ],
    )
    def k(g_hbm, src_hbm, dst_hbm, out_hbm, isrc_v, idst_v, bufs, gsem,
          csem, acc_sh):
        c = lax.axis_index("c")
        s = lax.axis_index("s")

        pltpu.sync_copy(g_hbm.at[pl.ds(s * RS, RS)],
                        acc_sh.at[pl.ds(s * RS, RS)])
        plsc.subcore_barrier()

        dummy = g_hbm.at[pl.ds(0, 64)]

        def gwait(sl):
            pltpu.make_async_copy(dummy, bufs[sl], gsem[sl]).wait()

        def cwait(sl):
            pltpu.make_async_copy(bufs[sl], acc_sh.at[idst_v.at[0]],
                                  csem[sl]).wait()

        def run(base_blocks, nphase, q):
            for phase in range(nphase):
                base = base_blocks + phase * q
                pltpu.sync_copy(src_hbm.at[pl.ds(base, q)],
                                isrc_v.at[pl.ds(0, q)])
                pltpu.sync_copy(dst_hbm.at[pl.ds(base, q)],
                                idst_v.at[pl.ds(0, q)])
                pltpu.async_copy(g_hbm.at[isrc_v.at[0]], bufs[0], gsem[0])
                pltpu.async_copy(g_hbm.at[isrc_v.at[1]], bufs[1], gsem[1])

                @pl.loop(0, q - 2, step=2)
                def _(j):
                    gwait(0)
                    pltpu.async_copy(bufs[0], acc_sh.at[idst_v.at[j]],
                                     csem[0], add=True)
                    gwait(1)
                    pltpu.async_copy(bufs[1], acc_sh.at[idst_v.at[j + 1]],
                                     csem[1], add=True)
                    cwait(0)
                    pltpu.async_copy(g_hbm.at[isrc_v.at[j + 2]], bufs[0],
                                     gsem[0])
                    cwait(1)
                    pltpu.async_copy(g_hbm.at[isrc_v.at[j + 3]], bufs[1],
                                     gsem[1])

                gwait(0)
                pltpu.async_copy(bufs[0], acc_sh.at[idst_v.at[q - 2]],
                                 csem[0], add=True)
                gwait(1)
                pltpu.async_copy(bufs[1], acc_sh.at[idst_v.at[q - 1]],
                                 csem[1], add=True)
                cwait(0)
                cwait(1)

        @pl.when(c == HFAST)
        def _():
            run(s * HF0, 4, HF0 // 4)

        @pl.when(c == 1 - HFAST)
        def _():
            run(16 * HF0 + s * HF1, 2, HF1 // 2)

        plsc.subcore_barrier()
        pltpu.sync_copy(acc_sh.at[pl.ds(s * RS, RS)],
                        out_hbm.at[c].at[pl.ds(s * RS, RS)])

    return k(g, src2d, dst2d)


def _sc_pair_gather(h, src2d, dst2d):
    @pl.kernel(
        out_type=(jax.ShapeDtypeStruct((EP, D), _f32),
                  jax.ShapeDtypeStruct((EP, D), _f32)),
        mesh=_mesh,
        scratch_types=[
            pltpu.VMEM((PQ, 128), jnp.int32),
            pltpu.VMEM((PQ, 128), jnp.int32),
            [pltpu.VMEM((128, D), _f32)] * 3,
            [pltpu.VMEM((128, D), _f32)] * 3,
            [pltpu.SemaphoreType.DMA] * 3,
            [pltpu.SemaphoreType.DMA] * 3,
        ],
    )
    def k(h_hbm, src_hbm, dst_hbm, hs_hbm, hd_hbm, isrc_v, idst_v,
          bs, bd, gsem, wsem):
        c = lax.axis_index("c")
        s = lax.axis_index("s")
        q = PQ

        dummy = h_hbm.at[pl.ds(0, 128)]

        def gwait(sl):
            pltpu.make_async_copy(dummy, bs[sl], gsem[sl]).wait()
            pltpu.make_async_copy(dummy, bd[sl], gsem[sl]).wait()

        def wwait(sl):
            pltpu.make_async_copy(bs[sl], hs_hbm.at[pl.ds(0, 128)],
                                  wsem[sl]).wait()
            pltpu.make_async_copy(bd[sl], hd_hbm.at[pl.ds(0, 128)],
                                  wsem[sl]).wait()

        def gstart(sl, j):
            pltpu.async_copy(h_hbm.at[isrc_v.at[j]], bs[sl], gsem[sl])
            pltpu.async_copy(h_hbm.at[idst_v.at[j]], bd[sl], gsem[sl])

        def wstart(sl, row0):
            pltpu.async_copy(bs[sl], hs_hbm.at[pl.ds(row0, 128)], wsem[sl])
            pltpu.async_copy(bd[sl], hd_hbm.at[pl.ds(row0, 128)], wsem[sl])

        def run(base_blocks, nphase):
          for phase in range(nphase):
            base = base_blocks + phase * q
            pltpu.sync_copy(src_hbm.at[pl.ds(base, q)], isrc_v)
            pltpu.sync_copy(dst_hbm.at[pl.ds(base, q)], idst_v)
            gstart(0, 0)
            gstart(1, 1)
            gwait(0)
            wstart(0, base * 128)
            gstart(2, 2)
            gwait(1)
            wstart(1, (base + 1) * 128)
            wwait(0)
            gstart(0, 3)

            nfull = (q - 8) // 3
            stop = 2 + 3 * nfull

            @pl.loop(2, stop, step=3)
            def _(j):
                for k3, sl in ((0, 2), (1, 0), (2, 1)):
                    jj = j + k3
                    s2 = (sl + 2) % 3
                    gwait(sl)
                    wstart(sl, (base + jj) * 128)
                    wwait(s2)
                    gstart(s2, jj + 2)

            for j0 in range(stop, q - 2):
                sl = j0 % 3
                s2 = (sl + 2) % 3
                gwait(sl)
                wstart(sl, (base + j0) * 128)
                wwait(s2)
                gstart(s2, j0 + 2)
            for j0 in (q - 2, q - 1):
                sl = j0 % 3
                gwait(sl)
                wstart(sl, (base + j0) * 128)
            for sl in range(3):
                wwait(sl)

        @pl.when(c == HFAST)
        def _():
            run(s * PF0, PF0 // PQ)

        @pl.when(c == 1 - HFAST)
        def _():
            run(16 * PF0 + s * PF1, PF1 // PQ)

    return k(h, src2d, dst2d)


_R = 2048


def _tc_scale0(xp, degp):
    def body(x_ref, d_ref, o_ref):
        deg = d_ref[0] + d_ref[1] - 1.0
        o_ref[...] = x_ref[...] * lax.rsqrt(deg)

    return pl.pallas_call(
        body,
        grid=(NP // _R,),
        in_specs=[pl.BlockSpec((_R, D), lambda i: (i, 0)),
                  pl.BlockSpec((2, _R, 1), lambda i: (0, i, 0))],
        out_specs=pl.BlockSpec((_R, D), lambda i: (i, 0)),
        out_shape=jax.ShapeDtypeStruct((NP, D), _f32),
    )(xp, degp)


def _tc_combine_mid(p, g, degp):
    def body(p_ref, g_ref, d_ref, o_ref):
        deg = d_ref[0] + d_ref[1] - 1.0
        o_ref[...] = (p_ref[0] + p_ref[1] - g_ref[...]) / deg

    return pl.pallas_call(
        body,
        grid=(NP // _R,),
        in_specs=[pl.BlockSpec((2, _R, D), lambda i: (0, i, 0)),
                  pl.BlockSpec((_R, D), lambda i: (i, 0)),
                  pl.BlockSpec((2, _R, 1), lambda i: (0, i, 0))],
        out_specs=pl.BlockSpec((_R, D), lambda i: (i, 0)),
        out_shape=jax.ShapeDtypeStruct((NP, D), _f32),
    )(p, g, degp)


def _tc_final(p, g, degp, W_sgc, b_sgc):
    def body(p_ref, g_ref, d_ref, w_ref, b_ref, o_ref):
        deg = d_ref[0] + d_ref[1] - 1.0
        hpre = (p_ref[0] + p_ref[1] - g_ref[...]) * lax.rsqrt(deg)
        o_ref[...] = jnp.dot(hpre, w_ref[...],
                             preferred_element_type=_f32) + b_ref[...]

    return pl.pallas_call(
        body,
        grid=(NP // _R,),
        in_specs=[pl.BlockSpec((2, _R, D), lambda i: (0, i, 0)),
                  pl.BlockSpec((_R, D), lambda i: (i, 0)),
                  pl.BlockSpec((2, _R, 1), lambda i: (0, i, 0)),
                  pl.BlockSpec((D, D), lambda i: (0, 0)),
                  pl.BlockSpec((1, D), lambda i: (0, 0))],
        out_specs=pl.BlockSpec((_R, D), lambda i: (i, 0)),
        out_shape=jax.ShapeDtypeStruct((NP, D), _f32),
    )(p, g, degp, W_sgc, b_sgc)


def _tc_mlp(hs, hd, W1, b1, W2, b2, W3, b3):
    def body(hs_ref, hd_ref, w1_ref, b1_ref, w2_ref, b2_ref, w3_ref, b3_ref,
             o_ref):
        z = hs_ref[...] * hd_ref[...]
        z = jnp.maximum(
            jnp.dot(z, w1_ref[...], preferred_element_type=_f32) + b1_ref[...],
            0.0)
        z = jnp.maximum(
            jnp.dot(z, w2_ref[...], preferred_element_type=_f32) + b2_ref[...],
            0.0)
        o_ref[...] = (jnp.dot(z, w3_ref[...], preferred_element_type=_f32)
                      + b3_ref[0, 0])

    return pl.pallas_call(
        body,
        grid=(-(-E // _R),),
        in_specs=[pl.BlockSpec((_R, D), lambda i: (i, 0)),
                  pl.BlockSpec((_R, D), lambda i: (i, 0)),
                  pl.BlockSpec((D, D), lambda i: (0, 0)),
                  pl.BlockSpec((1, D), lambda i: (0, 0)),
                  pl.BlockSpec((D, D), lambda i: (0, 0)),
                  pl.BlockSpec((1, D), lambda i: (0, 0)),
                  pl.BlockSpec((D, 1), lambda i: (0, 0)),
                  pl.BlockSpec((1, 1), lambda i: (0, 0))],
        out_specs=pl.BlockSpec((_R, 1), lambda i: (i, 0)),
        out_shape=jax.ShapeDtypeStruct((E, 1), _f32),
    )(hs, hd, W1, b1, W2, b2, W3, b3)


def _pad_idx(row, fill):
    pad = jnp.full((EP - E,), fill, jnp.int32)
    return jnp.concatenate([row, pad]).reshape(EB, 128)


def kernel(x, pos_edge_index, neg_edge_index, W_sgc, b_sgc, W1, b1, W2, b2,
           W3, b3):
    xp = jnp.pad(x, ((0, NP - N), (0, 0)))
    psrc = _pad_idx(pos_edge_index[0], 0)
    pdst = _pad_idx(pos_edge_index[1], N)
    nsrc = _pad_idx(neg_edge_index[0], 0)
    ndst = _pad_idx(neg_edge_index[1], N)

    psrc64 = psrc.reshape(EB64, 64)
    pdst64 = pdst.reshape(EB64, 64)

    degp = _sc_degree(pdst).reshape(2, NP, 1)
    g = _tc_scale0(xp, degp)
    h = None
    for hop in range(3):
        p = _sc_hop(g, psrc64, pdst64)
        if hop < 2:
            g = _tc_combine_mid(p, g, degp)
        else:
            h = _tc_final(p, g, degp, W_sgc, b_sgc.reshape(1, D))

    hs_p, hd_p = _sc_pair_gather(h, psrc, pdst)
    hs_n, hd_n = _sc_pair_gather(h, nsrc, ndst)
    b1r, b2r, b3r = b1.reshape(1, D), b2.reshape(1, D), b3.reshape(1, 1)
    op = _tc_mlp(hs_p, hd_p, W1, b1r, W2, b2r, W3, b3r)
    on = _tc_mlp(hs_n, hd_n, W1, b1r, W2, b2r, W3, b3r)
    return (op, on)
